# baseline (speedup 1.0000x reference)
import functools
import math

import jax
import jax.numpy as jnp
from jax import lax
from jax.experimental import pallas as pl
from jax.experimental.pallas import tpu as pltpu
from jax.experimental.pallas import tpu_sc as plsc

_N_TABLE = 4000
_N_NON = 6000
_N_NODES = 10000
_N_COLS = 26
_DIM = 128
_N_HEADS = 8
_HEAD_DIM = 16
_N_EDGES = 320000
_FFN_DIM = 256

_TR = 16
_BLK = 1000
_NC = 2
_NS = 16
_NW = _NC * _NS
_EPW = _N_EDGES // _NW
_CH = 128
_CHD = 80
_NCHUNKD = _EPW // _CHD
_EPWP = 10240
_NCHUNK = _EPWP // _CH
_N_PAD = 10240
_RPS = _N_PAD // _NS


def _ln(x, g, b):
    m = jnp.mean(x, axis=-1, keepdims=True)
    xc = x - m
    v = jnp.mean(xc * xc, axis=-1, keepdims=True)
    return xc * lax.rsqrt(v + 1e-5) * g + b


def _table_body(x_ref, wqkv_ref, wo_ref, g1_ref, b1_ref,
                wf1_ref, bf1_ref, wf2_ref, bf2_ref, g2_ref, b2_ref, out_ref):
    x = x_ref[...].reshape(_TR * _N_COLS, _DIM)
    qkv = jnp.dot(x, wqkv_ref[...], preferred_element_type=jnp.float32)
    q = qkv[:, :_DIM].reshape(_TR, _N_COLS, _DIM)
    k = qkv[:, _DIM:2 * _DIM].reshape(_TR, _N_COLS, _DIM)
    v = qkv[:, 2 * _DIM:].reshape(_TR, _N_COLS, _DIM)
    heads = []
    for h in range(_N_HEADS):
        lo = h * _HEAD_DIM
        qh = q[:, :, lo:lo + _HEAD_DIM]
        kh = k[:, :, lo:lo + _HEAD_DIM]
        vh = v[:, :, lo:lo + _HEAD_DIM]
        s = lax.dot_general(qh, kh, (((2,), (2,)), ((0,), (0,))),
                            preferred_element_type=jnp.float32)
        e = jnp.exp(s)
        vp = jnp.concatenate(
            [vh, jnp.ones((_TR, _N_COLS, _HEAD_DIM), jnp.float32)], axis=2)
        u = lax.dot_general(e, vp, (((2,), (1,)), ((0,), (0,))),
                            preferred_element_type=jnp.float32)
        heads.append(u[:, :, :_HEAD_DIM] / u[:, :, _HEAD_DIM:_HEAD_DIM + 1])
    o = jnp.concatenate(heads, axis=-1).reshape(_TR * _N_COLS, _DIM)
    o = jnp.dot(o, wo_ref[...], preferred_element_type=jnp.float32)
    x1 = _ln(x + o, g1_ref[...], b1_ref[...])
    f = jax.nn.gelu(jnp.dot(x1, wf1_ref[...], preferred_element_type=jnp.float32)
                    + bf1_ref[...])
    f = jnp.dot(f, wf2_ref[...], preferred_element_type=jnp.float32) + bf2_ref[...]
    x2 = _ln(x1 + f, g2_ref[...], b2_ref[...])
    out_ref[...] = jnp.mean(x2.reshape(_TR, _N_COLS, _DIM), axis=1)


def _table_encode(table, Wq, Wk, Wv, Wo, g1, b1, Wf1, bf1, Wf2, bf2, g2, b2):
    full = lambda shape: pl.BlockSpec(shape, lambda i: tuple(0 for _ in shape))
    wqkv = jnp.concatenate(
        [Wq * (1.0 / math.sqrt(_HEAD_DIM)), Wk, Wv], axis=1)
    return pl.pallas_call(
        _table_body,
        grid=(_N_TABLE // _TR,),
        in_specs=[
            pl.BlockSpec((_TR, _N_COLS, _DIM), lambda i: (i, 0, 0)),
            full((_DIM, 3 * _DIM)),
            full((_DIM, _DIM)),
            full((1, _DIM)), full((1, _DIM)),
            full((_DIM, _FFN_DIM)), full((1, _FFN_DIM)),
            full((_FFN_DIM, _DIM)), full((1, _DIM)),
            full((1, _DIM)), full((1, _DIM)),
        ],
        out_specs=pl.BlockSpec((_TR, _DIM), lambda i: (i, 0)),
        out_shape=jax.ShapeDtypeStruct((_N_TABLE, _DIM), jnp.float32),
    )(table, wqkv, Wo, g1, b1, Wf1, bf1, Wf2, bf2, g2, b2)


def _deg_terms(c0_ref, c1_ref):
    deg = c0_ref[...][:, 0:1] + c1_ref[...][:, 0:1] + 1.0
    return deg, lax.rsqrt(deg)


def _gcn_pre_body(x_ref, w_ref, c0_ref, c1_ref, hs_ref, self_ref):
    deg, r = _deg_terms(c0_ref, c1_ref)
    hw = jnp.dot(x_ref[...], w_ref[...], preferred_element_type=jnp.float32)
    hs_ref[...] = hw * r
    self_ref[...] = hw / deg


def _gcn_mid_body(a0_ref, a1_ref, self_ref, c0_ref, c1_ref, b1_ref, w2_ref,
                  hs2_ref, self2_ref):
    deg, r = _deg_terms(c0_ref, c1_ref)
    h = r * (a0_ref[...] + a1_ref[...]) + self_ref[...] + b1_ref[...]
    h = jnp.maximum(h, 0.0)
    hw = jnp.dot(h, w2_ref[...], preferred_element_type=jnp.float32)
    hs2_ref[...] = hw * r
    self2_ref[...] = hw / deg


def _gcn_out_body(a0_ref, a1_ref, self_ref, c0_ref, c1_ref, b2_ref, out_ref):
    deg, r = _deg_terms(c0_ref, c1_ref)
    out_ref[...] = r * (a0_ref[...] + a1_ref[...]) + self_ref[...] + b2_ref[...]


def _rows(shape):
    return pl.BlockSpec(shape, lambda i: (i,) + tuple(0 for _ in shape[1:]))


def _bcast(shape):
    return pl.BlockSpec(shape, lambda i: tuple(0 for _ in shape))


def _gcn_pre(nodes, W, c0, c1):
    return pl.pallas_call(
        _gcn_pre_body,
        grid=(_N_NODES // _BLK,),
        in_specs=[_rows((_BLK, _DIM)), _bcast((_DIM, _DIM)),
                  _rows((_BLK, 16)), _rows((_BLK, 16))],
        out_specs=[_rows((_BLK, _DIM)), _rows((_BLK, _DIM))],
        out_shape=[jax.ShapeDtypeStruct((_N_NODES, _DIM), jnp.float32),
                   jax.ShapeDtypeStruct((_N_NODES, _DIM), jnp.float32)],
    )(nodes, W, c0, c1)


def _gcn_mid(a0, a1, self1, c0, c1, b1, W2):
    return pl.pallas_call(
        _gcn_mid_body,
        grid=(_N_NODES // _BLK,),
        in_specs=[_rows((_BLK, _DIM)), _rows((_BLK, _DIM)), _rows((_BLK, _DIM)),
                  _rows((_BLK, 16)), _rows((_BLK, 16)),
                  _bcast((1, _DIM)), _bcast((_DIM, _DIM))],
        out_specs=[_rows((_BLK, _DIM)), _rows((_BLK, _DIM))],
        out_shape=[jax.ShapeDtypeStruct((_N_NODES, _DIM), jnp.float32),
                   jax.ShapeDtypeStruct((_N_NODES, _DIM), jnp.float32)],
    )(a0, a1, self1, c0, c1, b1, W2)


def _gcn_out(a0, a1, self2, c0, c1, b2):
    return pl.pallas_call(
        _gcn_out_body,
        grid=(_N_TABLE // _BLK,),
        in_specs=[_rows((_BLK, _DIM)), _rows((_BLK, _DIM)), _rows((_BLK, _DIM)),
                  _rows((_BLK, 16)), _rows((_BLK, 16)), _bcast((1, _DIM))],
        out_specs=_rows((_BLK, _DIM)),
        out_shape=jax.ShapeDtypeStruct((_N_TABLE, _DIM), jnp.float32),
    )(a0, a1, self2, c0, c1, b2)


def _sc_mesh():
    return plsc.VectorSubcoreMesh(core_axis_name="c", subcore_axis_name="s",
                                  num_cores=_NC, num_subcores=_NS)


def _sc_deg_body(dst_hbm, z16_hbm, out_hbm, didx, ones_v, acc):
    cid = lax.axis_index("c")
    sid = lax.axis_index("s")
    wid = sid * _NC + cid
    one = jnp.ones((16,), jnp.float32)
    for i in range(_CHD):
        ones_v[i, :] = one
    rbase = sid * _RPS
    pltpu.sync_copy(z16_hbm.at[pl.ds(rbase, _RPS)], acc.at[pl.ds(rbase, _RPS)])
    plsc.subcore_barrier()
    ebase = wid * _EPW

    def _chunk(i, carry):
        off = ebase + i * _CHD
        pltpu.sync_copy(dst_hbm.at[pl.ds(off, _CHD)], didx)
        pltpu.sync_copy(ones_v, acc.at[didx], add=True)
        return carry

    lax.fori_loop(0, _NCHUNKD, _chunk, 0)
    plsc.subcore_barrier()
    pltpu.sync_copy(acc.at[pl.ds(rbase, _RPS)],
                    out_hbm.at[cid, pl.ds(rbase, _RPS)])


def _sc_degrees(dst):
    z16 = jnp.zeros((_N_PAD, 16), jnp.float32)
    call = pl.kernel(
        _sc_deg_body,
        out_type=jax.ShapeDtypeStruct((_NC, _N_PAD, 16), jnp.float32),
        mesh=_sc_mesh(),
        scratch_types=[
            pltpu.VMEM((_CHD,), jnp.int32),
            pltpu.VMEM((_CHD, 16), jnp.float32),
            pltpu.VMEM_SHARED((_N_PAD, 16), jnp.float32),
        ],
    )
    return call(dst, z16)


def _sc_agg_body(hs_hbm, src_hbm, dst_hbm, z_hbm, out_hbm, sidx, didx, rows, acc):
    cid = lax.axis_index("c")
    sid = lax.axis_index("s")
    wid = sid * _NC + cid
    rbase = sid * _RPS
    pltpu.sync_copy(z_hbm.at[pl.ds(rbase, _RPS)], acc.at[pl.ds(rbase, _RPS)])
    plsc.subcore_barrier()
    ebase = wid * _EPW

    def _chunk(i, carry):
        off = ebase + i * _CHD
        pltpu.sync_copy(src_hbm.at[pl.ds(off, _CHD)], sidx)
        pltpu.sync_copy(dst_hbm.at[pl.ds(off, _CHD)], didx)
        pltpu.sync_copy(hs_hbm.at[sidx], rows)
        pltpu.sync_copy(rows, acc.at[didx], add=True)
        return carry

    lax.fori_loop(0, _NCHUNKD, _chunk, 0)
    plsc.subcore_barrier()
    pltpu.sync_copy(acc.at[pl.ds(rbase, _RPS)],
                    out_hbm.at[cid, pl.ds(rbase, _RPS)])


def _sc_aggregate(hs, src, dst):
    z = jnp.zeros((_N_PAD, _DIM), jnp.float32)
    call = pl.kernel(
        _sc_agg_body,
        out_type=jax.ShapeDtypeStruct((_NC, _N_PAD, _DIM), jnp.float32),
        mesh=_sc_mesh(),
        scratch_types=[
            pltpu.VMEM((_CHD,), jnp.int32),
            pltpu.VMEM((_CHD,), jnp.int32),
            pltpu.VMEM((_CHD, _DIM), jnp.float32),
            pltpu.VMEM_SHARED((_N_PAD, _DIM), jnp.float32),
        ],
    )
    return call(hs, src, dst, z)


def kernel(table, non_table, adj, Wq, Wk, Wv, Wo, ln1_g, ln1_b, Wf1, bf1,
           Wf2, bf2, ln2_g, ln2_b, Wg1, bg1, Wg2, bg2):
    npad = _EPWP - _EPW
    spread = (jnp.arange(npad, dtype=jnp.int32) % 240)[None, :]
    src2 = adj[0].astype(jnp.int32).reshape(_NW, _EPW)
    dst2 = adj[1].astype(jnp.int32).reshape(_NW, _EPW)
    srcp = jnp.concatenate(
        [src2, jnp.broadcast_to(spread, (_NW, npad))], axis=1).reshape(-1)
    dstp = jnp.concatenate(
        [dst2, jnp.broadcast_to(_N_NODES + spread, (_NW, npad))],
        axis=1).reshape(-1)

    src_flat = adj[0].astype(jnp.int32)
    dst_flat = adj[1].astype(jnp.int32)
    cnt = _sc_degrees(dst_flat)
    c0 = cnt[0]
    c1 = cnt[1]

    t_emb = _table_encode(table, Wq, Wk, Wv, Wo,
                          ln1_g.reshape(1, -1), ln1_b.reshape(1, -1),
                          Wf1, bf1.reshape(1, -1), Wf2, bf2.reshape(1, -1),
                          ln2_g.reshape(1, -1), ln2_b.reshape(1, -1))
    nodes = jnp.concatenate([t_emb, non_table], axis=0)

    hs1, self1 = _gcn_pre(nodes, Wg1, c0, c1)
    agg1 = _sc_aggregate(hs1, src_flat, dst_flat)
    hs2, self2 = _gcn_mid(agg1[0], agg1[1], self1, c0, c1,
                          bg1.reshape(1, -1), Wg2)
    agg2 = _sc_aggregate(hs2, src_flat, dst_flat)
    return _gcn_out(agg2[0], agg2[1], self2, c0, c1, bg2.reshape(1, -1))


# speedup vs baseline: 1.2966x; 1.2966x over previous
import functools
import math

import jax
import jax.numpy as jnp
from jax import lax
from jax.experimental import pallas as pl
from jax.experimental.pallas import tpu as pltpu
from jax.experimental.pallas import tpu_sc as plsc

_N_TABLE = 4000
_N_NON = 6000
_N_NODES = 10000
_N_COLS = 26
_DIM = 128
_N_HEADS = 8
_HEAD_DIM = 16
_N_EDGES = 320000
_FFN_DIM = 256

_TR = 16
_BLK = 1000
_NC = 2
_NS = 16
_NW = _NC * _NS
_EPW = _N_EDGES // _NW
_CH = 128
_CHD = 80
_NCHUNKD = _EPW // _CHD
_EPWP = 10240
_NCHUNK = _EPWP // _CH
_N_PAD = 10240
_RPS = _N_PAD // _NS


def _ln(x, g, b):
    m = jnp.mean(x, axis=-1, keepdims=True)
    xc = x - m
    v = jnp.mean(xc * xc, axis=-1, keepdims=True)
    return xc * lax.rsqrt(v + 1e-5) * g + b


def _table_body(x_ref, wqkv_ref, wo_ref, g1_ref, b1_ref,
                wf1_ref, bf1_ref, wf2_ref, bf2_ref, g2_ref, b2_ref, out_ref):
    x = x_ref[...].reshape(_TR * _N_COLS, _DIM)
    qkv = jnp.dot(x, wqkv_ref[...], preferred_element_type=jnp.float32)
    q = qkv[:, :_DIM].reshape(_TR, _N_COLS, _DIM)
    k = qkv[:, _DIM:2 * _DIM].reshape(_TR, _N_COLS, _DIM)
    v = qkv[:, 2 * _DIM:].reshape(_TR, _N_COLS, _DIM)
    heads = []
    for h in range(_N_HEADS):
        lo = h * _HEAD_DIM
        qh = q[:, :, lo:lo + _HEAD_DIM]
        kh = k[:, :, lo:lo + _HEAD_DIM]
        vh = v[:, :, lo:lo + _HEAD_DIM]
        s = lax.dot_general(qh, kh, (((2,), (2,)), ((0,), (0,))),
                            preferred_element_type=jnp.float32)
        e = jnp.exp(s)
        vp = jnp.concatenate(
            [vh, jnp.ones((_TR, _N_COLS, _HEAD_DIM), jnp.float32)], axis=2)
        u = lax.dot_general(e, vp, (((2,), (1,)), ((0,), (0,))),
                            preferred_element_type=jnp.float32)
        heads.append(u[:, :, :_HEAD_DIM] / u[:, :, _HEAD_DIM:_HEAD_DIM + 1])
    o = jnp.concatenate(heads, axis=-1).reshape(_TR * _N_COLS, _DIM)
    o = jnp.dot(o, wo_ref[...], preferred_element_type=jnp.float32)
    x1 = _ln(x + o, g1_ref[...], b1_ref[...])
    f = jax.nn.gelu(jnp.dot(x1, wf1_ref[...], preferred_element_type=jnp.float32)
                    + bf1_ref[...])
    f = jnp.dot(f, wf2_ref[...], preferred_element_type=jnp.float32) + bf2_ref[...]
    x2 = _ln(x1 + f, g2_ref[...], b2_ref[...])
    out_ref[...] = jnp.mean(x2.reshape(_TR, _N_COLS, _DIM), axis=1)


def _table_encode(table, Wq, Wk, Wv, Wo, g1, b1, Wf1, bf1, Wf2, bf2, g2, b2):
    full = lambda shape: pl.BlockSpec(shape, lambda i: tuple(0 for _ in shape))
    wqkv = jnp.concatenate(
        [Wq * (1.0 / math.sqrt(_HEAD_DIM)), Wk, Wv], axis=1)
    return pl.pallas_call(
        _table_body,
        grid=(_N_TABLE // _TR,),
        in_specs=[
            pl.BlockSpec((_TR, _N_COLS, _DIM), lambda i: (i, 0, 0)),
            full((_DIM, 3 * _DIM)),
            full((_DIM, _DIM)),
            full((1, _DIM)), full((1, _DIM)),
            full((_DIM, _FFN_DIM)), full((1, _FFN_DIM)),
            full((_FFN_DIM, _DIM)), full((1, _DIM)),
            full((1, _DIM)), full((1, _DIM)),
        ],
        out_specs=pl.BlockSpec((_TR, _DIM), lambda i: (i, 0)),
        out_shape=jax.ShapeDtypeStruct((_N_TABLE, _DIM), jnp.float32),
    )(table, wqkv, Wo, g1, b1, Wf1, bf1, Wf2, bf2, g2, b2)


def _deg_terms(c0_ref, c1_ref):
    deg = c0_ref[...][:, 0:1] + c1_ref[...][:, 0:1] + 1.0
    return deg, lax.rsqrt(deg)


def _gcn_pre_body(x_ref, w_ref, c0_ref, c1_ref, hs_ref, self_ref):
    deg, r = _deg_terms(c0_ref, c1_ref)
    hw = jnp.dot(x_ref[...], w_ref[...], preferred_element_type=jnp.float32)
    hs_ref[...] = hw * r
    self_ref[...] = hw / deg


def _gcn_mid_body(a0_ref, a1_ref, self_ref, c0_ref, c1_ref, b1_ref, w2_ref,
                  hs2_ref, self2_ref):
    deg, r = _deg_terms(c0_ref, c1_ref)
    h = r * (a0_ref[...] + a1_ref[...]) + self_ref[...] + b1_ref[...]
    h = jnp.maximum(h, 0.0)
    hw = jnp.dot(h, w2_ref[...], preferred_element_type=jnp.float32)
    hs2_ref[...] = hw * r
    self2_ref[...] = hw / deg


def _gcn_out_body(a0_ref, a1_ref, self_ref, c0_ref, c1_ref, b2_ref, out_ref):
    deg, r = _deg_terms(c0_ref, c1_ref)
    out_ref[...] = r * (a0_ref[...] + a1_ref[...]) + self_ref[...] + b2_ref[...]


def _rows(shape):
    return pl.BlockSpec(shape, lambda i: (i,) + tuple(0 for _ in shape[1:]))


def _bcast(shape):
    return pl.BlockSpec(shape, lambda i: tuple(0 for _ in shape))


def _gcn_pre(nodes, W, c0, c1):
    return pl.pallas_call(
        _gcn_pre_body,
        grid=(_N_NODES // _BLK,),
        in_specs=[_rows((_BLK, _DIM)), _bcast((_DIM, _DIM)),
                  _rows((_BLK, 16)), _rows((_BLK, 16))],
        out_specs=[_rows((_BLK, _DIM)), _rows((_BLK, _DIM))],
        out_shape=[jax.ShapeDtypeStruct((_N_NODES, _DIM), jnp.float32),
                   jax.ShapeDtypeStruct((_N_NODES, _DIM), jnp.float32)],
    )(nodes, W, c0, c1)


def _gcn_mid(a0, a1, self1, c0, c1, b1, W2):
    return pl.pallas_call(
        _gcn_mid_body,
        grid=(_N_NODES // _BLK,),
        in_specs=[_rows((_BLK, _DIM)), _rows((_BLK, _DIM)), _rows((_BLK, _DIM)),
                  _rows((_BLK, 16)), _rows((_BLK, 16)),
                  _bcast((1, _DIM)), _bcast((_DIM, _DIM))],
        out_specs=[_rows((_BLK, _DIM)), _rows((_BLK, _DIM))],
        out_shape=[jax.ShapeDtypeStruct((_N_NODES, _DIM), jnp.float32),
                   jax.ShapeDtypeStruct((_N_NODES, _DIM), jnp.float32)],
    )(a0, a1, self1, c0, c1, b1, W2)


def _gcn_out(a0, a1, self2, c0, c1, b2):
    return pl.pallas_call(
        _gcn_out_body,
        grid=(_N_TABLE // _BLK,),
        in_specs=[_rows((_BLK, _DIM)), _rows((_BLK, _DIM)), _rows((_BLK, _DIM)),
                  _rows((_BLK, 16)), _rows((_BLK, 16)), _bcast((1, _DIM))],
        out_specs=_rows((_BLK, _DIM)),
        out_shape=jax.ShapeDtypeStruct((_N_TABLE, _DIM), jnp.float32),
    )(a0, a1, self2, c0, c1, b2)


def _sc_mesh():
    return plsc.VectorSubcoreMesh(core_axis_name="c", subcore_axis_name="s",
                                  num_cores=_NC, num_subcores=_NS)


def _sc_deg_body(dst_hbm, z16_hbm, out_hbm, didx, ones_v, acc):
    cid = lax.axis_index("c")
    sid = lax.axis_index("s")
    wid = sid * _NC + cid
    one = jnp.ones((16,), jnp.float32)
    for i in range(_CHD):
        ones_v[i, :] = one
    rbase = sid * _RPS
    pltpu.sync_copy(z16_hbm.at[pl.ds(rbase, _RPS)], acc.at[pl.ds(rbase, _RPS)])
    plsc.subcore_barrier()
    ebase = wid * _EPW

    def _chunk(i, carry):
        off = ebase + i * _CHD
        pltpu.sync_copy(dst_hbm.at[pl.ds(off, _CHD)], didx)
        pltpu.sync_copy(ones_v, acc.at[didx], add=True)
        return carry

    lax.fori_loop(0, _NCHUNKD, _chunk, 0)
    plsc.subcore_barrier()
    pltpu.sync_copy(acc.at[pl.ds(rbase, _RPS)],
                    out_hbm.at[cid, pl.ds(rbase, _RPS)])


def _sc_degrees(dst):
    z16 = jnp.zeros((_N_PAD, 16), jnp.float32)
    call = pl.kernel(
        _sc_deg_body,
        out_type=jax.ShapeDtypeStruct((_NC, _N_PAD, 16), jnp.float32),
        mesh=_sc_mesh(),
        scratch_types=[
            pltpu.VMEM((_CHD,), jnp.int32),
            pltpu.VMEM((_CHD, 16), jnp.float32),
            pltpu.VMEM_SHARED((_N_PAD, 16), jnp.float32),
        ],
    )
    return call(dst, z16)


def _sc_agg_body(hs_hbm, srcp_hbm, dstp_hbm, z_hbm, out_hbm,
                 sidx_a, didx_a, sidx_b, didx_b, rows0, rows1, acc,
                 sem0, sem1, sem_a, sem_b):
    cid = lax.axis_index("c")
    sid = lax.axis_index("s")
    wid = sid * _NC + cid
    rbase = sid * _RPS
    ebase = wid * _EPWP

    def _fetch(buf_s, buf_d, c, sem):
        off = ebase + c * _CH
        pltpu.async_copy(srcp_hbm.at[pl.ds(off, _CH)], buf_s, sem)
        pltpu.async_copy(dstp_hbm.at[pl.ds(off, _CH)], buf_d, sem)

    def _fetch_wait(buf_s, buf_d, sem):
        pltpu.make_async_copy(srcp_hbm.at[pl.ds(0, _CH)], buf_s, sem).wait()
        pltpu.make_async_copy(dstp_hbm.at[pl.ds(0, _CH)], buf_d, sem).wait()

    pltpu.sync_copy(z_hbm.at[pl.ds(rbase, _RPS)], acc.at[pl.ds(rbase, _RPS)])
    plsc.subcore_barrier()

    _fetch(sidx_a, didx_a, 0, sem_a)
    _fetch_wait(sidx_a, didx_a, sem_a)
    pltpu.async_copy(hs_hbm.at[sidx_a], rows0, sem0)
    _fetch(sidx_b, didx_b, 1, sem_b)

    def _pair(g, carry):
        ca = 2 * g
        _fetch_wait(sidx_b, didx_b, sem_b)
        pltpu.async_copy(hs_hbm.at[sidx_b], rows1, sem1)
        pltpu.make_async_copy(hs_hbm.at[sidx_a], rows0, sem0).wait()
        pltpu.sync_copy(rows0, acc.at[didx_a], add=True)

        @pl.when(ca + 2 < _NCHUNK)
        def _():
            _fetch(sidx_a, didx_a, ca + 2, sem_a)
            _fetch_wait(sidx_a, didx_a, sem_a)
            pltpu.async_copy(hs_hbm.at[sidx_a], rows0, sem0)

        pltpu.make_async_copy(hs_hbm.at[sidx_b], rows1, sem1).wait()
        pltpu.sync_copy(rows1, acc.at[didx_b], add=True)

        @pl.when(ca + 3 < _NCHUNK)
        def _():
            _fetch(sidx_b, didx_b, ca + 3, sem_b)

        return carry

    lax.fori_loop(0, _NCHUNK // 2, _pair, 0)
    plsc.subcore_barrier()
    pltpu.sync_copy(acc.at[pl.ds(rbase, _RPS)],
                    out_hbm.at[cid, pl.ds(rbase, _RPS)])


def _sc_aggregate(hs, srcp, dstp):
    z = jnp.zeros((_N_PAD, _DIM), jnp.float32)
    call = pl.kernel(
        _sc_agg_body,
        out_type=jax.ShapeDtypeStruct((_NC, _N_PAD, _DIM), jnp.float32),
        mesh=_sc_mesh(),
        scratch_types=[
            pltpu.VMEM((_CH,), jnp.int32),
            pltpu.VMEM((_CH,), jnp.int32),
            pltpu.VMEM((_CH,), jnp.int32),
            pltpu.VMEM((_CH,), jnp.int32),
            pltpu.VMEM((_CH, _DIM), jnp.float32),
            pltpu.VMEM((_CH, _DIM), jnp.float32),
            pltpu.VMEM_SHARED((_N_PAD, _DIM), jnp.float32),
            pltpu.SemaphoreType.DMA,
            pltpu.SemaphoreType.DMA,
            pltpu.SemaphoreType.DMA,
            pltpu.SemaphoreType.DMA,
        ],
    )
    return call(hs, srcp, dstp, z)


def kernel(table, non_table, adj, Wq, Wk, Wv, Wo, ln1_g, ln1_b, Wf1, bf1,
           Wf2, bf2, ln2_g, ln2_b, Wg1, bg1, Wg2, bg2):
    npad = _EPWP - _EPW
    spread = (jnp.arange(npad, dtype=jnp.int32) % 240)[None, :]
    src2 = adj[0].astype(jnp.int32).reshape(_NW, _EPW)
    dst2 = adj[1].astype(jnp.int32).reshape(_NW, _EPW)
    srcp = jnp.concatenate(
        [src2, jnp.broadcast_to(spread, (_NW, npad))], axis=1).reshape(-1)
    dstp = jnp.concatenate(
        [dst2, jnp.broadcast_to(_N_NODES + spread, (_NW, npad))],
        axis=1).reshape(-1)

    cnt = _sc_degrees(adj[1].astype(jnp.int32))
    c0 = cnt[0]
    c1 = cnt[1]

    t_emb = _table_encode(table, Wq, Wk, Wv, Wo,
                          ln1_g.reshape(1, -1), ln1_b.reshape(1, -1),
                          Wf1, bf1.reshape(1, -1), Wf2, bf2.reshape(1, -1),
                          ln2_g.reshape(1, -1), ln2_b.reshape(1, -1))
    nodes = jnp.concatenate([t_emb, non_table], axis=0)

    hs1, self1 = _gcn_pre(nodes, Wg1, c0, c1)
    agg1 = _sc_aggregate(hs1, srcp, dstp)
    hs2, self2 = _gcn_mid(agg1[0], agg1[1], self1, c0, c1,
                          bg1.reshape(1, -1), Wg2)
    agg2 = _sc_aggregate(hs2, srcp, dstp)
    return _gcn_out(agg2[0], agg2[1], self2, c0, c1, bg2.reshape(1, -1))
